# baseline (speedup 1.0000x reference)
import functools

import jax
import jax.numpy as jnp
from jax import lax
from jax.experimental import pallas as pl
from jax.experimental.pallas import tpu as pltpu
from jax.experimental.pallas import tpu_sc as plsc

N = 10000
NH = N // 2
E = 320000
D = 128
NC = 2
NS = 16
NW = NC * NS
CH = E // NW
B = 32
GRP = 8
CAPQ = 4480
KMAX = CAPQ - GRP * B - 16
HP = 5120
XR = N // NS
AR = HP // NS

_mesh = lambda: plsc.VectorSubcoreMesh(core_axis_name="c", subcore_axis_name="s")

_sc_params = pltpu.CompilerParams(needs_layout_passes=False,
                                  use_tc_tiling_on_sc=False)


def _compact_body(src_h, dst_h, et_h,
                  s_out, d_out, cnts_h, c0_h, c1_h,
                  sv, dv, tv, s00, d00, s01, d01, s10, d10, s11, d11,
                  c0, c1, cv):
    cid = lax.axis_index("c")
    sid = lax.axis_index("s")
    wid = cid * NS + sid
    base = wid * CH
    pltpu.sync_copy(src_h.at[pl.ds(base, CH)], sv)
    pltpu.sync_copy(dst_h.at[pl.ds(base, CH)], dv)
    pltpu.sync_copy(et_h.at[pl.ds(base, CH)], tv)

    zf = jnp.zeros((16,), jnp.float32)

    def zbody(i, carry):
        c0[pl.ds(i * 16, 16)] = zf
        c1[pl.ds(i * 16, 16)] = zf
        return carry

    lax.fori_loop(0, N // 16, zbody, 0)

    ones = jnp.ones((16,), jnp.float32)
    sbuf = ((s00, d00), (s01, d01), (s10, d10), (s11, d11))

    def ebody(i, ks):
        s = sv[pl.ds(i * 16, 16)]
        d = dv[pl.ds(i * 16, 16)]
        t = tv[pl.ds(i * 16, 16)]
        m0 = t == 0
        m1 = t == 1
        plsc.addupdate_scatter(c0, [s], ones, mask=m0)
        plsc.addupdate_scatter(c1, [s], ones, mask=m1)
        hi = s >= NH
        lo = jnp.logical_not(hi)
        s_hi = s - NH
        masks = (m0 & lo, m0 & hi, m1 & lo, m1 & hi)
        svals = (s, s_hi, s, s_hi)
        out = []
        for q in range(4):
            kq = ks[q]
            plsc.store_compressed(sbuf[q][0].at[pl.ds(kq, 16)], svals[q],
                                  mask=masks[q])
            plsc.store_compressed(sbuf[q][1].at[pl.ds(kq, 16)], d,
                                  mask=masks[q])
            kq = kq + jnp.sum(masks[q].astype(jnp.int32))
            out.append(jnp.minimum(kq, KMAX))
        return tuple(out)

    z = jnp.int32(0)
    ks = lax.fori_loop(0, CH // 16, ebody, (z, z, z, z))

    lanes = lax.iota(jnp.int32, 16)
    nbs = []
    for q in range(4):
        kq = ks[q]
        for u in range(GRP * B // 16):
            spread = (wid * (GRP * B // 16) + u) * 16
            dummy_s = NH + ((spread + lanes) % (HP - NH))
            dummy_d = (spread * 7 + lanes) % N
            sbuf[q][0][pl.ds(kq + u * 16, 16)] = dummy_s
            sbuf[q][1][pl.ds(kq + u * 16, 16)] = dummy_d
        nbs.append(GRP * ((kq + (GRP * B - 1)) // (GRP * B)))

    cv[...] = (jnp.where(lanes == 0, nbs[0], 0)
               + jnp.where(lanes == 1, nbs[1], 0)
               + jnp.where(lanes == 2, nbs[2], 0)
               + jnp.where(lanes == 3, nbs[3], 0))

    for q in range(4):
        pltpu.sync_copy(sbuf[q][0], s_out.at[wid, q])
        pltpu.sync_copy(sbuf[q][1], d_out.at[wid, q])
    pltpu.sync_copy(cv, cnts_h.at[wid])
    pltpu.sync_copy(c0, c0_h.at[wid])
    pltpu.sync_copy(c1, c1_h.at[wid])


_compact = pl.kernel(
    _compact_body,
    out_type=(
        jax.ShapeDtypeStruct((NW, 4, CAPQ), jnp.int32),
        jax.ShapeDtypeStruct((NW, 4, CAPQ), jnp.int32),
        jax.ShapeDtypeStruct((NW, 16), jnp.int32),
        jax.ShapeDtypeStruct((NW, N), jnp.float32),
        jax.ShapeDtypeStruct((NW, N), jnp.float32),
    ),
    mesh=_mesh(),
    scratch_types=[
        pltpu.VMEM((CH,), jnp.int32),
        pltpu.VMEM((CH,), jnp.int32),
        pltpu.VMEM((CH,), jnp.int32),
        pltpu.VMEM((CAPQ,), jnp.int32),
        pltpu.VMEM((CAPQ,), jnp.int32),
        pltpu.VMEM((CAPQ,), jnp.int32),
        pltpu.VMEM((CAPQ,), jnp.int32),
        pltpu.VMEM((CAPQ,), jnp.int32),
        pltpu.VMEM((CAPQ,), jnp.int32),
        pltpu.VMEM((CAPQ,), jnp.int32),
        pltpu.VMEM((CAPQ,), jnp.int32),
        pltpu.VMEM((N,), jnp.float32),
        pltpu.VMEM((N,), jnp.float32),
        pltpu.VMEM((16,), jnp.int32),
    ],
    compiler_params=_sc_params,
)


def _agg_body(slot, feat_h, zero_h, s_h, d_h, cnts_h, out_h,
              sidx, didx, r0, r1, cva, cvb,
              g0, g1, i0, i1, i2, i3, i4, i5, i6, i7,
              semx, semz, semc, xsp, acc):
    cid = lax.axis_index("c")
    sid = lax.axis_index("s")
    rows = (r0, r1)
    gsem = (g0, g1)
    isem = (i0, i1, i2, i3, i4, i5, i6, i7)
    q = 2 * slot + cid

    xb = sid * XR
    dx = pltpu.async_copy(feat_h.at[pl.ds(xb, XR)], xsp.at[pl.ds(xb, XR)],
                          semx)
    ab = sid * AR
    dz = pltpu.async_copy(zero_h, acc.at[pl.ds(ab, AR)], semz)
    wa = 2 * sid
    wb = 2 * sid + 1
    dca = pltpu.async_copy(cnts_h.at[wa], cva, semc)
    dcb = pltpu.async_copy(cnts_h.at[wb], cvb, semc)
    dca.wait()
    dcb.wait()
    lanes = lax.iota(jnp.int32, 16)
    nba = jnp.sum(jnp.where(lanes == q, cva[...], 0))
    nbb = jnp.sum(jnp.where(lanes == q, cvb[...], 0))
    dx.wait()
    dz.wait()
    plsc.subcore_barrier()

    def run_chunk(w, nb):
        def fire_idx(j, sl):
            pltpu.async_copy(s_h.at[w, q, pl.ds(j * B, B)], sidx.at[sl],
                             isem[sl])
            pltpu.async_copy(d_h.at[w, q, pl.ds(j * B, B)], didx.at[sl],
                             isem[sl])

        def wait_idx(j, sl):
            pltpu.make_async_copy(s_h.at[w, q, pl.ds(j * B, B)],
                                  sidx.at[sl], isem[sl]).wait()
            pltpu.make_async_copy(d_h.at[w, q, pl.ds(j * B, B)],
                                  didx.at[sl], isem[sl]).wait()

        def fire_gather(sl, rb):
            pltpu.async_copy(xsp.at[didx.at[sl]], rows[rb], gsem[rb])

        def wait_gather(sl, rb):
            pltpu.make_async_copy(xsp.at[didx.at[sl]], rows[rb],
                                  gsem[rb]).wait()

        for j in range(GRP):
            @pl.when(j < nb)
            def _pidx():
                fire_idx(j, j)
        for j in range(2):
            @pl.when(j < nb)
            def _pg():
                wait_idx(j, j)
                fire_gather(j, j)

        def grp(g, carry):
            j0 = GRP * g
            for b in range(GRP):
                j = j0 + b
                wait_gather(b, b % 2)
                pltpu.sync_copy(rows[b % 2], acc.at[sidx.at[b]], add=True)

                @pl.when(j + 2 < nb)
                def _nxt():
                    wait_idx(j + 2, (b + 2) % GRP)
                    fire_gather((b + 2) % GRP, b % 2)

                @pl.when(j + GRP < nb)
                def _ri():
                    fire_idx(j + GRP, b)
            return carry

        lax.fori_loop(0, nb // GRP, grp, 0)

    run_chunk(wa, nba)
    run_chunk(wb, nbb)
    plsc.subcore_barrier()

    ob = cid * NH + sid * AR

    @pl.when(sid < NS - 1)
    def _full():
        pltpu.sync_copy(acc.at[pl.ds(ab, AR)], out_h.at[pl.ds(ob, AR)])

    @pl.when(sid == NS - 1)
    def _tail():
        pltpu.sync_copy(acc.at[pl.ds(ab, NH - (NS - 1) * AR)],
                        out_h.at[pl.ds(ob, NH - (NS - 1) * AR)])


def _make_agg(slot):
    return pl.kernel(
        functools.partial(_agg_body, slot),
        out_type=jax.ShapeDtypeStruct((N, D), jnp.float32),
        mesh=_mesh(),
        scratch_types=[
            pltpu.VMEM((GRP, B), jnp.int32),
            pltpu.VMEM((GRP, B), jnp.int32),
            pltpu.VMEM((B, D), jnp.float32),
            pltpu.VMEM((B, D), jnp.float32),
            pltpu.VMEM((16,), jnp.int32),
            pltpu.VMEM((16,), jnp.int32),
            pltpu.SemaphoreType.DMA,
            pltpu.SemaphoreType.DMA,
            pltpu.SemaphoreType.DMA,
            pltpu.SemaphoreType.DMA,
            pltpu.SemaphoreType.DMA,
            pltpu.SemaphoreType.DMA,
            pltpu.SemaphoreType.DMA,
            pltpu.SemaphoreType.DMA,
            pltpu.SemaphoreType.DMA,
            pltpu.SemaphoreType.DMA,
            pltpu.SemaphoreType.DMA,
            pltpu.SemaphoreType.DMA,
            pltpu.SemaphoreType.DMA,
            pltpu.VMEM_SHARED((N, D), jnp.float32),
            pltpu.VMEM_SHARED((HP, D), jnp.float32),
        ],
        compiler_params=_sc_params,
    )


_agg0 = _make_agg(0)
_agg1 = _make_agg(1)

BK = 2048


def _blk_cnt(cp):
    return jnp.maximum(jnp.sum(cp[...], axis=0), 1.0)


def _layer_body(a, cp, x, w, r, b, o):
    cnt = _blk_cnt(cp)
    agg = a[...] / cnt[:, None]
    h = (jnp.dot(agg, w[...], preferred_element_type=jnp.float32)
         + jnp.dot(x[...], r[...], preferred_element_type=jnp.float32)
         + b[...])
    o[...] = jnp.maximum(h, 0.0)


def _final_body(a, cp, x, w, r, b, wl, bl, o):
    cnt = _blk_cnt(cp)
    agg = a[...] / cnt[:, None]
    h = (jnp.dot(agg, w[...], preferred_element_type=jnp.float32)
         + jnp.dot(x[...], r[...], preferred_element_type=jnp.float32)
         + b[...])
    h = jnp.maximum(h, 0.0)
    o[...] = jnp.dot(h, wl[...], preferred_element_type=jnp.float32) + bl[...]


def _row_spec():
    return pl.BlockSpec((BK, D), lambda i: (i, 0))


def _full_spec():
    return pl.BlockSpec((D, D), lambda i: (0, 0))


def _bias_spec():
    return pl.BlockSpec((1, D), lambda i: (0, 0))


def _layer(agg, cntp, x, w, root, b):
    return pl.pallas_call(
        _layer_body,
        grid=(pl.cdiv(N, BK),),
        in_specs=[
            _row_spec(),
            pl.BlockSpec((NW, BK), lambda i: (0, i)),
            _row_spec(), _full_spec(), _full_spec(), _bias_spec(),
        ],
        out_specs=_row_spec(),
        out_shape=jax.ShapeDtypeStruct((N, D), jnp.float32),
    )(agg, cntp, x, w, root, b)


def _final(agg, cntp, x, w, root, b, wl, bl):
    return pl.pallas_call(
        _final_body,
        grid=(pl.cdiv(N, BK),),
        in_specs=[
            _row_spec(),
            pl.BlockSpec((NW, BK), lambda i: (0, i)),
            _row_spec(), _full_spec(), _full_spec(), _bias_spec(),
            _full_spec(), _bias_spec(),
        ],
        out_specs=_row_spec(),
        out_shape=jax.ShapeDtypeStruct((N, D), jnp.float32),
    )(agg, cntp, x, w, root, b, wl, bl)


def kernel(x, edge_index, edge_type, W1, root1, b1, W2, root2, b2, Wl, bl):
    src = edge_index[0]
    dst = edge_index[1]
    s_lists, d_lists, cnts, c0p, c1p = _compact(src, dst, edge_type)
    zeros = jnp.zeros((AR, D), jnp.float32)
    agg0 = _agg0(x, zeros, s_lists, d_lists, cnts)
    h1 = _layer(agg0, c0p, x, W1[0], root1, b1.reshape(1, D))
    agg1 = _agg1(h1, zeros, s_lists, d_lists, cnts)
    out = _final(agg1, c1p, h1, W2[1], root2, b2.reshape(1, D),
                 Wl, bl.reshape(1, D))
    return out


# speedup vs baseline: 1.0570x; 1.0570x over previous
import functools

import jax
import jax.numpy as jnp
from jax import lax
from jax.experimental import pallas as pl
from jax.experimental.pallas import tpu as pltpu
from jax.experimental.pallas import tpu_sc as plsc

N = 10000
NH = N // 2
E = 320000
D = 128
NC = 2
NS = 16
NW = NC * NS
CH = E // NW
B = 32
GRP = 4
CAPQ = 4480
KMAX = CAPQ - GRP * B - 16
HP = 5120
XR = N // NS
AR = HP // NS
SH = 14
DMASK = (1 << SH) - 1

_mesh = lambda: plsc.VectorSubcoreMesh(core_axis_name="c", subcore_axis_name="s")

_sc_params = pltpu.CompilerParams(needs_layout_passes=False,
                                  use_tc_tiling_on_sc=False)


def _compact_body(src_h, dst_h, et_h,
                  p_out, cnts_h, c0_h, c1_h,
                  sv, dv, tv, p00, p01, p10, p11, c0, c1, cv,
                  semin, semout):
    cid = lax.axis_index("c")
    sid = lax.axis_index("s")
    wid = cid * NS + sid
    base = wid * CH
    ds = pltpu.async_copy(src_h.at[pl.ds(base, CH)], sv, semin)
    dd = pltpu.async_copy(dst_h.at[pl.ds(base, CH)], dv, semin)
    dt = pltpu.async_copy(et_h.at[pl.ds(base, CH)], tv, semin)

    zf = jnp.zeros((16,), jnp.float32)

    def zbody(i, carry):
        c0[pl.ds(i * 16, 16)] = zf
        c1[pl.ds(i * 16, 16)] = zf
        return carry

    lax.fori_loop(0, N // 16, zbody, 0)
    ds.wait()
    dd.wait()
    dt.wait()

    ones = jnp.ones((16,), jnp.float32)
    pbuf = (p00, p01, p10, p11)

    def ebody(i, ks):
        s = sv[pl.ds(i * 16, 16)]
        d = dv[pl.ds(i * 16, 16)]
        t = tv[pl.ds(i * 16, 16)]
        m0 = t == 0
        m1 = t == 1
        plsc.addupdate_scatter(c0, [s], ones, mask=m0)
        plsc.addupdate_scatter(c1, [s], ones, mask=m1)
        hi = s >= NH
        lo = jnp.logical_not(hi)
        p_lo = (s << SH) | d
        p_hi = ((s - NH) << SH) | d
        masks = (m0 & lo, m0 & hi, m1 & lo, m1 & hi)
        pvals = (p_lo, p_hi, p_lo, p_hi)
        out = []
        for q in range(4):
            kq = ks[q]
            plsc.store_compressed(pbuf[q].at[pl.ds(kq, 16)], pvals[q],
                                  mask=masks[q])
            kq = kq + jnp.sum(masks[q].astype(jnp.int32))
            out.append(jnp.minimum(kq, KMAX))
        return tuple(out)

    z = jnp.int32(0)
    ks = lax.fori_loop(0, CH // 16, ebody, (z, z, z, z))

    lanes = lax.iota(jnp.int32, 16)
    nbs = []
    for q in range(4):
        kq = ks[q]
        for u in range(GRP * B // 16):
            spread = (wid * (GRP * B // 16) + u) * 16
            dummy_s = NH + ((spread + lanes) % (HP - NH))
            dummy_d = (spread * 7 + lanes) % N
            pbuf[q][pl.ds(kq + u * 16, 16)] = (dummy_s << SH) | dummy_d
        nbs.append(GRP * ((kq + (GRP * B - 1)) // (GRP * B)))

    cv[...] = (jnp.where(lanes == 0, nbs[0], 0)
               + jnp.where(lanes == 1, nbs[1], 0)
               + jnp.where(lanes == 2, nbs[2], 0)
               + jnp.where(lanes == 3, nbs[3], 0))

    outs = [pltpu.async_copy(pbuf[q], p_out.at[wid, q], semout)
            for q in range(4)]
    outs.append(pltpu.async_copy(cv, cnts_h.at[wid], semout))
    outs.append(pltpu.async_copy(c0, c0_h.at[wid], semout))
    outs.append(pltpu.async_copy(c1, c1_h.at[wid], semout))
    for o in outs:
        o.wait()


_compact = pl.kernel(
    _compact_body,
    out_type=(
        jax.ShapeDtypeStruct((NW, 4, CAPQ), jnp.int32),
        jax.ShapeDtypeStruct((NW, 16), jnp.int32),
        jax.ShapeDtypeStruct((NW, N), jnp.float32),
        jax.ShapeDtypeStruct((NW, N), jnp.float32),
    ),
    mesh=_mesh(),
    scratch_types=[
        pltpu.VMEM((CH,), jnp.int32),
        pltpu.VMEM((CH,), jnp.int32),
        pltpu.VMEM((CH,), jnp.int32),
        pltpu.VMEM((CAPQ,), jnp.int32),
        pltpu.VMEM((CAPQ,), jnp.int32),
        pltpu.VMEM((CAPQ,), jnp.int32),
        pltpu.VMEM((CAPQ,), jnp.int32),
        pltpu.VMEM((N,), jnp.float32),
        pltpu.VMEM((N,), jnp.float32),
        pltpu.VMEM((16,), jnp.int32),
        pltpu.SemaphoreType.DMA,
        pltpu.SemaphoreType.DMA,
    ],
    compiler_params=_sc_params,
)


def _agg_body(slot, feat_h, zero_h, p_h, cnts_h, out_h,
              pidx, sidx, didx, r0, r1, cva, cvb,
              g0, g1, i0, i1, i2, i3,
              semx, semz, semc, xsp, acc):
    cid = lax.axis_index("c")
    sid = lax.axis_index("s")
    rows = (r0, r1)
    gsem = (g0, g1)
    isem = (i0, i1, i2, i3)
    q = 2 * slot + cid

    xb = sid * XR
    dx = pltpu.async_copy(feat_h.at[pl.ds(xb, XR)], xsp.at[pl.ds(xb, XR)],
                          semx)
    ab = sid * AR
    dz = pltpu.async_copy(zero_h, acc.at[pl.ds(ab, AR)], semz)
    wa = 2 * sid
    wb = 2 * sid + 1
    dca = pltpu.async_copy(cnts_h.at[wa], cva, semc)
    dcb = pltpu.async_copy(cnts_h.at[wb], cvb, semc)
    dca.wait()
    dcb.wait()
    lanes = lax.iota(jnp.int32, 16)
    nba = jnp.sum(jnp.where(lanes == q, cva[...], 0))
    nbb = jnp.sum(jnp.where(lanes == q, cvb[...], 0))
    dx.wait()
    dz.wait()
    plsc.subcore_barrier()

    def run_chunk(w, nb):
        def fire_idx(j, sl):
            pltpu.async_copy(p_h.at[w, q, pl.ds(j * B, B)], pidx.at[sl],
                             isem[sl])

        def unpack(sl):
            pltpu.make_async_copy(p_h.at[w, q, pl.ds(0, B)],
                                  pidx.at[sl], isem[sl]).wait()
            for h in range(B // 16):
                v = pidx[sl, pl.ds(h * 16, 16)]
                didx[sl, pl.ds(h * 16, 16)] = v & DMASK
                sidx[sl, pl.ds(h * 16, 16)] = v >> SH

        def fire_gather(sl, rb):
            pltpu.async_copy(xsp.at[didx.at[sl]], rows[rb], gsem[rb])

        def wait_gather(rb):
            pltpu.make_async_copy(xsp.at[didx.at[0]], rows[rb],
                                  gsem[rb]).wait()

        for j in range(GRP):
            @pl.when(j < nb)
            def _pidx():
                fire_idx(j, j)
        for j in range(2):
            @pl.when(j < nb)
            def _pg():
                unpack(j)
                fire_gather(j, j)

        def grp(g, carry):
            j0 = GRP * g
            for b in range(GRP):
                j = j0 + b
                wait_gather(b % 2)
                pltpu.sync_copy(rows[b % 2], acc.at[sidx.at[b]], add=True)

                @pl.when(j + 2 < nb)
                def _nxt():
                    unpack((b + 2) % GRP)
                    fire_gather((b + 2) % GRP, b % 2)

                @pl.when(j + GRP < nb)
                def _ri():
                    fire_idx(j + GRP, b)
            return carry

        lax.fori_loop(0, nb // GRP, grp, 0)

    run_chunk(wa, nba)
    run_chunk(wb, nbb)
    plsc.subcore_barrier()

    ob = cid * NH + sid * AR

    @pl.when(sid < NS - 1)
    def _full():
        pltpu.sync_copy(acc.at[pl.ds(ab, AR)], out_h.at[pl.ds(ob, AR)])

    @pl.when(sid == NS - 1)
    def _tail():
        pltpu.sync_copy(acc.at[pl.ds(ab, NH - (NS - 1) * AR)],
                        out_h.at[pl.ds(ob, NH - (NS - 1) * AR)])


def _make_agg(slot):
    return pl.kernel(
        functools.partial(_agg_body, slot),
        out_type=jax.ShapeDtypeStruct((N, D), jnp.float32),
        mesh=_mesh(),
        scratch_types=[
            pltpu.VMEM((GRP, B), jnp.int32),
            pltpu.VMEM((GRP, B), jnp.int32),
            pltpu.VMEM((GRP, B), jnp.int32),
            pltpu.VMEM((B, D), jnp.float32),
            pltpu.VMEM((B, D), jnp.float32),
            pltpu.VMEM((16,), jnp.int32),
            pltpu.VMEM((16,), jnp.int32),
            pltpu.SemaphoreType.DMA,
            pltpu.SemaphoreType.DMA,
            pltpu.SemaphoreType.DMA,
            pltpu.SemaphoreType.DMA,
            pltpu.SemaphoreType.DMA,
            pltpu.SemaphoreType.DMA,
            pltpu.SemaphoreType.DMA,
            pltpu.SemaphoreType.DMA,
            pltpu.SemaphoreType.DMA,
            pltpu.VMEM_SHARED((N, D), jnp.float32),
            pltpu.VMEM_SHARED((HP, D), jnp.float32),
        ],
        compiler_params=_sc_params,
    )


_agg0 = _make_agg(0)
_agg1 = _make_agg(1)

BK = 2048


def _blk_cnt(cp):
    return jnp.maximum(jnp.sum(cp[...], axis=0), 1.0)


def _layer_body(a, cp, x, w, r, b, o):
    cnt = _blk_cnt(cp)
    agg = a[...] / cnt[:, None]
    h = (jnp.dot(agg, w[...], preferred_element_type=jnp.float32)
         + jnp.dot(x[...], r[...], preferred_element_type=jnp.float32)
         + b[...])
    o[...] = jnp.maximum(h, 0.0)


def _final_body(a, cp, x, w, r, b, wl, bl, o):
    cnt = _blk_cnt(cp)
    agg = a[...] / cnt[:, None]
    h = (jnp.dot(agg, w[...], preferred_element_type=jnp.float32)
         + jnp.dot(x[...], r[...], preferred_element_type=jnp.float32)
         + b[...])
    h = jnp.maximum(h, 0.0)
    o[...] = jnp.dot(h, wl[...], preferred_element_type=jnp.float32) + bl[...]


def _row_spec():
    return pl.BlockSpec((BK, D), lambda i: (i, 0))


def _full_spec():
    return pl.BlockSpec((D, D), lambda i: (0, 0))


def _bias_spec():
    return pl.BlockSpec((1, D), lambda i: (0, 0))


def _layer(agg, cntp, x, w, root, b):
    return pl.pallas_call(
        _layer_body,
        grid=(pl.cdiv(N, BK),),
        in_specs=[
            _row_spec(),
            pl.BlockSpec((NW, BK), lambda i: (0, i)),
            _row_spec(), _full_spec(), _full_spec(), _bias_spec(),
        ],
        out_specs=_row_spec(),
        out_shape=jax.ShapeDtypeStruct((N, D), jnp.float32),
    )(agg, cntp, x, w, root, b)


def _final(agg, cntp, x, w, root, b, wl, bl):
    return pl.pallas_call(
        _final_body,
        grid=(pl.cdiv(N, BK),),
        in_specs=[
            _row_spec(),
            pl.BlockSpec((NW, BK), lambda i: (0, i)),
            _row_spec(), _full_spec(), _full_spec(), _bias_spec(),
            _full_spec(), _bias_spec(),
        ],
        out_specs=_row_spec(),
        out_shape=jax.ShapeDtypeStruct((N, D), jnp.float32),
    )(agg, cntp, x, w, root, b, wl, bl)


def kernel(x, edge_index, edge_type, W1, root1, b1, W2, root2, b2, Wl, bl):
    src = edge_index[0]
    dst = edge_index[1]
    p_lists, cnts, c0p, c1p = _compact(src, dst, edge_type)
    zeros = jnp.zeros((AR, D), jnp.float32)
    agg0 = _agg0(x, zeros, p_lists, cnts)
    h1 = _layer(agg0, c0p, x, W1[0], root1, b1.reshape(1, D))
    agg1 = _agg1(h1, zeros, p_lists, cnts)
    out = _final(agg1, c1p, h1, W2[1], root2, b2.reshape(1, D),
                 Wl, bl.reshape(1, D))
    return out


# speedup vs baseline: 1.1033x; 1.0439x over previous
import functools

import jax
import jax.numpy as jnp
from jax import lax
from jax.experimental import pallas as pl
from jax.experimental.pallas import tpu as pltpu
from jax.experimental.pallas import tpu_sc as plsc

N = 10000
NH = N // 2
E = 320000
D = 128
NC = 2
NS = 16
NW = NC * NS
CH = E // NW
B = 24
GRP = 6
CAPQ = 4480
KMAX = CAPQ - GRP * B - 16
HP = 5120
XR = N // NS
AR = HP // NS
SH = 14
DMASK = (1 << SH) - 1

_mesh = lambda: plsc.VectorSubcoreMesh(core_axis_name="c", subcore_axis_name="s")

_sc_params = pltpu.CompilerParams(needs_layout_passes=False,
                                  use_tc_tiling_on_sc=False)


def _compact_body(src_h, dst_h, et_h,
                  p_out, cnts_h, c0_h, c1_h,
                  sv, dv, tv, p00, p01, p10, p11, c0, c1, cv,
                  semin, semout):
    cid = lax.axis_index("c")
    sid = lax.axis_index("s")
    wid = cid * NS + sid
    base = wid * CH
    ds = pltpu.async_copy(src_h.at[pl.ds(base, CH)], sv, semin)
    dd = pltpu.async_copy(dst_h.at[pl.ds(base, CH)], dv, semin)
    dt = pltpu.async_copy(et_h.at[pl.ds(base, CH)], tv, semin)

    zf = jnp.zeros((16,), jnp.float32)

    def zbody(i, carry):
        c0[pl.ds(i * 16, 16)] = zf
        c1[pl.ds(i * 16, 16)] = zf
        return carry

    lax.fori_loop(0, N // 16, zbody, 0)
    ds.wait()
    dd.wait()
    dt.wait()

    ones = jnp.ones((16,), jnp.float32)
    pbuf = (p00, p01, p10, p11)

    def ebody(i, ks):
        s = sv[pl.ds(i * 16, 16)]
        d = dv[pl.ds(i * 16, 16)]
        t = tv[pl.ds(i * 16, 16)]
        m0 = t == 0
        m1 = t == 1
        plsc.addupdate_scatter(c0, [s], ones, mask=m0)
        plsc.addupdate_scatter(c1, [s], ones, mask=m1)
        hi = s >= NH
        lo = jnp.logical_not(hi)
        p_lo = (s << SH) | d
        p_hi = ((s - NH) << SH) | d
        masks = (m0 & lo, m0 & hi, m1 & lo, m1 & hi)
        pvals = (p_lo, p_hi, p_lo, p_hi)
        out = []
        for q in range(4):
            kq = ks[q]
            plsc.store_compressed(pbuf[q].at[pl.ds(kq, 16)], pvals[q],
                                  mask=masks[q])
            kq = kq + jnp.sum(masks[q].astype(jnp.int32))
            out.append(jnp.minimum(kq, KMAX))
        return tuple(out)

    z = jnp.int32(0)
    ks = lax.fori_loop(0, CH // 16, ebody, (z, z, z, z))

    lanes = lax.iota(jnp.int32, 16)
    nbs = []
    for q in range(4):
        kq = ks[q]
        for u in range(GRP * B // 16):
            spread = (wid * (GRP * B // 16) + u) * 16
            dummy_s = NH + ((spread + lanes) % (HP - NH))
            dummy_d = (spread * 7 + lanes) % N
            pbuf[q][pl.ds(kq + u * 16, 16)] = (dummy_s << SH) | dummy_d
        nbs.append(GRP * ((kq + (GRP * B - 1)) // (GRP * B)))

    cv[...] = (jnp.where(lanes == 0, nbs[0], 0)
               + jnp.where(lanes == 1, nbs[1], 0)
               + jnp.where(lanes == 2, nbs[2], 0)
               + jnp.where(lanes == 3, nbs[3], 0))

    outs = [pltpu.async_copy(pbuf[q], p_out.at[wid, q], semout)
            for q in range(4)]
    outs.append(pltpu.async_copy(cv, cnts_h.at[wid], semout))
    outs.append(pltpu.async_copy(c0, c0_h.at[wid], semout))
    outs.append(pltpu.async_copy(c1, c1_h.at[wid], semout))
    for o in outs:
        o.wait()


_compact = pl.kernel(
    _compact_body,
    out_type=(
        jax.ShapeDtypeStruct((NW, 4, CAPQ), jnp.int32),
        jax.ShapeDtypeStruct((NW, 16), jnp.int32),
        jax.ShapeDtypeStruct((NW, N), jnp.float32),
        jax.ShapeDtypeStruct((NW, N), jnp.float32),
    ),
    mesh=_mesh(),
    scratch_types=[
        pltpu.VMEM((CH,), jnp.int32),
        pltpu.VMEM((CH,), jnp.int32),
        pltpu.VMEM((CH,), jnp.int32),
        pltpu.VMEM((CAPQ,), jnp.int32),
        pltpu.VMEM((CAPQ,), jnp.int32),
        pltpu.VMEM((CAPQ,), jnp.int32),
        pltpu.VMEM((CAPQ,), jnp.int32),
        pltpu.VMEM((N,), jnp.float32),
        pltpu.VMEM((N,), jnp.float32),
        pltpu.VMEM((16,), jnp.int32),
        pltpu.SemaphoreType.DMA,
        pltpu.SemaphoreType.DMA,
    ],
    compiler_params=_sc_params,
)


def _agg_body(slot, feat_h, zero_h, p_h, cnts_h, out_h,
              pidx, sidx, didx, r0, r1, r2, cva, cvb,
              g0, g1, g2, s0, s1, s2, i0, i1, i2, i3, i4, i5,
              semx, semz, semc, xsp, acc):
    cid = lax.axis_index("c")
    sid = lax.axis_index("s")
    rows = (r0, r1, r2)
    gsem = (g0, g1, g2)
    ssem = (s0, s1, s2)
    isem = (i0, i1, i2, i3, i4, i5)
    q = 2 * slot + cid

    xb = sid * XR
    dx = pltpu.async_copy(feat_h.at[pl.ds(xb, XR)], xsp.at[pl.ds(xb, XR)],
                          semx)
    ab = sid * AR
    dz = pltpu.async_copy(zero_h, acc.at[pl.ds(ab, AR)], semz)
    wa = 2 * sid
    wb = 2 * sid + 1
    dca = pltpu.async_copy(cnts_h.at[wa], cva, semc)
    dcb = pltpu.async_copy(cnts_h.at[wb], cvb, semc)
    dca.wait()
    dcb.wait()
    lanes = lax.iota(jnp.int32, 16)
    nba = jnp.sum(jnp.where(lanes == q, cva[...], 0))
    nbb = jnp.sum(jnp.where(lanes == q, cvb[...], 0))
    dx.wait()
    dz.wait()
    plsc.subcore_barrier()

    def run_chunk(w, nb):
        def fire_idx(j, sl):
            pltpu.async_copy(p_h.at[w, q, pl.ds(j * B, B)], pidx.at[sl],
                             isem[sl])

        def unpack(sl):
            pltpu.make_async_copy(p_h.at[w, q, pl.ds(0, B)],
                                  pidx.at[sl], isem[sl]).wait()
            for off in (0, B - 16):
                v = pidx[sl, pl.ds(off, 16)]
                didx[sl, pl.ds(off, 16)] = v & DMASK
                sidx[sl, pl.ds(off, 16)] = v >> SH

        def fire_gather(sl, rb):
            pltpu.async_copy(xsp.at[didx.at[sl]], rows[rb], gsem[rb])

        def wait_gather(rb):
            pltpu.make_async_copy(xsp.at[didx.at[0]], rows[rb],
                                  gsem[rb]).wait()

        def fire_scatter(sl, rb):
            pltpu.async_copy(rows[rb], acc.at[sidx.at[sl]], ssem[rb],
                             add=True)

        def drain_scatter(rb):
            pltpu.make_async_copy(rows[rb], acc.at[sidx.at[0]],
                                  ssem[rb]).wait()

        for j in range(GRP):
            @pl.when(j < nb)
            def _pidx():
                fire_idx(j, j)
        for j in range(2):
            @pl.when(j < nb)
            def _pg():
                unpack(j)
                fire_gather(j, j)

        def block(j, b, first):
            wait_gather(b % 3)
            fire_scatter(b, b % 3)

            @pl.when(j + 2 < nb)
            def _nxt():
                unpack((b + 2) % GRP)
                if not (first and b == 0):
                    drain_scatter((b + 2) % 3)
                fire_gather((b + 2) % GRP, (b + 2) % 3)

            @pl.when(j + GRP < nb)
            def _ri():
                fire_idx(j + GRP, b)

        for b in range(GRP):
            @pl.when(b < nb)
            def _p0():
                block(b, b, True)

        def grp(g, carry):
            j0 = GRP * g
            for b in range(GRP):
                block(j0 + b, b, False)
            return carry

        lax.fori_loop(1, nb // GRP, grp, 0)

        @pl.when(nb > 0)
        def _drain():
            drain_scatter(2)
            drain_scatter(1)
            drain_scatter(0)

    run_chunk(wa, nba)
    run_chunk(wb, nbb)
    plsc.subcore_barrier()

    ob = cid * NH + sid * AR

    @pl.when(sid < NS - 1)
    def _full():
        pltpu.sync_copy(acc.at[pl.ds(ab, AR)], out_h.at[pl.ds(ob, AR)])

    @pl.when(sid == NS - 1)
    def _tail():
        pltpu.sync_copy(acc.at[pl.ds(ab, NH - (NS - 1) * AR)],
                        out_h.at[pl.ds(ob, NH - (NS - 1) * AR)])


def _make_agg(slot):
    return pl.kernel(
        functools.partial(_agg_body, slot),
        out_type=jax.ShapeDtypeStruct((N, D), jnp.float32),
        mesh=_mesh(),
        scratch_types=[
            pltpu.VMEM((GRP, B), jnp.int32),
            pltpu.VMEM((GRP, B), jnp.int32),
            pltpu.VMEM((GRP, B), jnp.int32),
            pltpu.VMEM((B, D), jnp.float32),
            pltpu.VMEM((B, D), jnp.float32),
            pltpu.VMEM((B, D), jnp.float32),
            pltpu.VMEM((16,), jnp.int32),
            pltpu.VMEM((16,), jnp.int32),
        ] + [pltpu.SemaphoreType.DMA] * 15 + [
            pltpu.VMEM_SHARED((N, D), jnp.float32),
            pltpu.VMEM_SHARED((HP, D), jnp.float32),
        ],
        compiler_params=_sc_params,
    )


_agg0 = _make_agg(0)
_agg1 = _make_agg(1)

BK = 2048


def _blk_cnt(cp):
    return jnp.maximum(jnp.sum(cp[...], axis=0), 1.0)


def _layer_body(a, cp, x, w, r, b, o):
    cnt = _blk_cnt(cp)
    agg = a[...] / cnt[:, None]
    h = (jnp.dot(agg, w[...], preferred_element_type=jnp.float32)
         + jnp.dot(x[...], r[...], preferred_element_type=jnp.float32)
         + b[...])
    o[...] = jnp.maximum(h, 0.0)


def _final_body(a, cp, x, w, r, b, wl, bl, o):
    cnt = _blk_cnt(cp)
    agg = a[...] / cnt[:, None]
    h = (jnp.dot(agg, w[...], preferred_element_type=jnp.float32)
         + jnp.dot(x[...], r[...], preferred_element_type=jnp.float32)
         + b[...])
    h = jnp.maximum(h, 0.0)
    o[...] = jnp.dot(h, wl[...], preferred_element_type=jnp.float32) + bl[...]


def _row_spec():
    return pl.BlockSpec((BK, D), lambda i: (i, 0))


def _full_spec():
    return pl.BlockSpec((D, D), lambda i: (0, 0))


def _bias_spec():
    return pl.BlockSpec((1, D), lambda i: (0, 0))


def _layer(agg, cntp, x, w, root, b):
    return pl.pallas_call(
        _layer_body,
        grid=(pl.cdiv(N, BK),),
        in_specs=[
            _row_spec(),
            pl.BlockSpec((NW, BK), lambda i: (0, i)),
            _row_spec(), _full_spec(), _full_spec(), _bias_spec(),
        ],
        out_specs=_row_spec(),
        out_shape=jax.ShapeDtypeStruct((N, D), jnp.float32),
    )(agg, cntp, x, w, root, b)


def _final(agg, cntp, x, w, root, b, wl, bl):
    return pl.pallas_call(
        _final_body,
        grid=(pl.cdiv(N, BK),),
        in_specs=[
            _row_spec(),
            pl.BlockSpec((NW, BK), lambda i: (0, i)),
            _row_spec(), _full_spec(), _full_spec(), _bias_spec(),
            _full_spec(), _bias_spec(),
        ],
        out_specs=_row_spec(),
        out_shape=jax.ShapeDtypeStruct((N, D), jnp.float32),
    )(agg, cntp, x, w, root, b, wl, bl)


def kernel(x, edge_index, edge_type, W1, root1, b1, W2, root2, b2, Wl, bl):
    src = edge_index[0]
    dst = edge_index[1]
    p_lists, cnts, c0p, c1p = _compact(src, dst, edge_type)
    zeros = jnp.zeros((AR, D), jnp.float32)
    agg0 = _agg0(x, zeros, p_lists, cnts)
    h1 = _layer(agg0, c0p, x, W1[0], root1, b1.reshape(1, D))
    agg1 = _agg1(h1, zeros, p_lists, cnts)
    out = _final(agg1, c1p, h1, W2[1], root2, b2.reshape(1, D),
                 Wl, bl.reshape(1, D))
    return out
